# baseline (speedup 1.0000x reference)
import functools
import jax
import jax.numpy as jnp
from jax import lax
from jax.experimental import pallas as pl
from jax.experimental.pallas import tpu as pltpu
from jax.experimental.pallas import tpu_sc as plsc

_K = 32
_N = 4096
_B = 4
_NB = 256
_NW = 32
_QPW = _N // _NW
_L = 16


def _make():
    mesh = plsc.VectorSubcoreMesh(core_axis_name="c", subcore_axis_name="s")

    @functools.partial(
        pl.kernel,
        out_type=jax.ShapeDtypeStruct((_NW, _L), jnp.float32),
        mesh=mesh,
        scratch_types=[
            pltpu.VMEM((_N,), jnp.float32),
            pltpu.VMEM((_N,), jnp.float32),
            pltpu.VMEM((_N,), jnp.float32),
            pltpu.VMEM((_N,), jnp.float32),
            pltpu.VMEM((_N,), jnp.float32),
            pltpu.VMEM((_N,), jnp.float32),
            pltpu.VMEM((_NB,), jnp.float32),
            pltpu.VMEM((_NB,), jnp.float32),
            pltpu.VMEM((_L,), jnp.float32),
        ],
        compiler_params=pltpu.CompilerParams(needs_layout_passes=False),
    )
    def sc_knn(soa_hbm, out_hbm, px_v, py_v, pz_v, fx_v, fy_v, fz_v,
               hist, fhist, out_v):
        wid = lax.axis_index("s") * 2 + lax.axis_index("c")
        iota = lax.iota(jnp.int32, _L)
        zeros_v = jnp.zeros((_L,), jnp.float32)
        ones_v = jnp.ones((_L,), jnp.float32)
        zidx = jnp.zeros((_L,), jnp.int32)
        kf = jnp.float32(_K)

        acc = zeros_v
        for b in range(_B):
            for i, buf in enumerate((px_v, py_v, pz_v, fx_v, fy_v, fz_v)):
                pltpu.sync_copy(soa_hbm.at[6 * b + i], buf)

            def per_query(qi, acc):
                q = wid * _QPW + qi
                qchunk = (q // _L) * _L
                qlane = q % _L

                def splat(buf):
                    v = buf[pl.ds(qchunk, _L)]
                    s = jnp.sum(jnp.where(iota == qlane, v, jnp.float32(0.0)))
                    return zeros_v + s

                qx = splat(px_v)
                qy = splat(py_v)
                qz = splat(pz_v)
                qfx = splat(fx_v)
                qfy = splat(fy_v)
                qfz = splat(fz_v)

                def zero_hist(i, carry):
                    hist[pl.ds(i * _L, _L)] = zeros_v
                    fhist[pl.ds(i * _L, _L)] = zeros_v
                    return carry

                lax.fori_loop(0, _NB // _L, zero_hist, 0, unroll=True)

                def sweep(c, carry):
                    base = c * _L
                    cx = px_v[pl.ds(base, _L)]
                    cy = py_v[pl.ds(base, _L)]
                    cz = pz_v[pl.ds(base, _L)]
                    dx = qx - cx
                    dy = qy - cy
                    dz = qz - cz
                    d = dx * dx + dy * dy + dz * dz

                    fx = fx_v[pl.ds(base, _L)]
                    fy = fy_v[pl.ds(base, _L)]
                    fz = fz_v[pl.ds(base, _L)]
                    gx = qfx - fx
                    gy = qfy - fy
                    gz = qfz - fz
                    sqf = gx * gx + gy * gy + gz * gz

                    sqfs = jnp.maximum(sqf, jnp.float32(1e-12))
                    i0 = plsc.bitcast(sqfs, jnp.int32)
                    i1 = jnp.int32(0x5F3759DF) - lax.shift_right_logical(i0, 1)
                    y = plsc.bitcast(i1, jnp.float32)
                    half = jnp.float32(0.5) * sqfs
                    for _ in range(3):
                        y = y * (jnp.float32(1.5) - half * y * y)
                    F = jnp.where(sqf > 0.0, sqfs * y, jnp.float32(0.0))

                    mask = d <= 1.0
                    bucket = jnp.minimum(
                        (d * jnp.float32(_NB)).astype(jnp.int32), _NB - 1)
                    plsc.addupdate_scatter(hist, [bucket], ones_v, mask=mask)
                    plsc.addupdate_scatter(fhist, [bucket], F, mask=mask)
                    return carry

                lax.fori_loop(0, _N // _L, sweep, 0)

                def select(i, carry):
                    run, acc = carry
                    h = hist[pl.ds(i * _L, _L)]
                    fh = fhist[pl.ds(i * _L, _L)]
                    incl = plsc.cumsum(h)
                    excl = incl - h + run
                    w = jnp.clip((kf - excl) / jnp.maximum(h, 1.0), 0.0, 1.0)
                    acc = acc + w * fh
                    run = run + jnp.sum(h)
                    return run, acc

                _, acc = lax.fori_loop(0, _NB // _L, select,
                                       (jnp.float32(0.0), acc))
                return acc

            acc = lax.fori_loop(0, _QPW, per_query, acc)

        out_v[...] = acc
        pltpu.sync_copy(out_v, out_hbm.at[wid])

    return sc_knn


def kernel(pc_source, pred_flow):
    soa = jnp.concatenate(
        [jnp.transpose(pc_source, (0, 2, 1)),
         jnp.transpose(pred_flow, (0, 2, 1))], axis=1)
    soa = soa.reshape(_B * 6, _N)
    out = _make()(soa)
    return jnp.sum(out) / jnp.float32(_B * _N * _K)


# speedup vs baseline: 1.0047x; 1.0047x over previous
import functools
import jax
import jax.numpy as jnp
from jax import lax
from jax.experimental import pallas as pl
from jax.experimental.pallas import tpu as pltpu
from jax.experimental.pallas import tpu_sc as plsc

_K = 32
_N = 4096
_B = 4
_NB = 256
_NW = 32
_QPW = _N // _NW
_L = 16


def _make():
    mesh = plsc.VectorSubcoreMesh(core_axis_name="c", subcore_axis_name="s")

    @functools.partial(
        pl.kernel,
        out_type=jax.ShapeDtypeStruct((_NW, _L), jnp.float32),
        mesh=mesh,
        scratch_types=[
            pltpu.VMEM((_N,), jnp.float32),
            pltpu.VMEM((_N,), jnp.float32),
            pltpu.VMEM((_N,), jnp.float32),
            pltpu.VMEM((_N,), jnp.float32),
            pltpu.VMEM((_N,), jnp.float32),
            pltpu.VMEM((_N,), jnp.float32),
            pltpu.VMEM((_NB,), jnp.float32),
            pltpu.VMEM((_NB,), jnp.float32),
            pltpu.VMEM((_L,), jnp.float32),
        ],
        compiler_params=pltpu.CompilerParams(needs_layout_passes=False),
    )
    def sc_knn(soa_hbm, out_hbm, px_v, py_v, pz_v, fx_v, fy_v, fz_v,
               hist, fhist, out_v):
        wid = lax.axis_index("s") * 2 + lax.axis_index("c")
        iota = lax.iota(jnp.int32, _L)
        zeros_v = jnp.zeros((_L,), jnp.float32)
        ones_v = jnp.ones((_L,), jnp.float32)
        zidx = jnp.zeros((_L,), jnp.int32)
        kf = jnp.float32(_K)

        acc = zeros_v
        for b in range(_B):
            for i, buf in enumerate((px_v, py_v, pz_v, fx_v, fy_v, fz_v)):
                pltpu.sync_copy(soa_hbm.at[6 * b + i], buf)

            def per_query(qi, acc):
                q = wid * _QPW + qi
                qchunk = (q // _L) * _L
                qlane = q % _L

                def splat(buf):
                    v = buf[pl.ds(qchunk, _L)]
                    s = jnp.sum(jnp.where(iota == qlane, v, jnp.float32(0.0)))
                    return zeros_v + s

                qx = splat(px_v)
                qy = splat(py_v)
                qz = splat(pz_v)
                qfx = splat(fx_v)
                qfy = splat(fy_v)
                qfz = splat(fz_v)

                def zero_hist(i, carry):
                    hist[pl.ds(i * _L, _L)] = zeros_v
                    fhist[pl.ds(i * _L, _L)] = zeros_v
                    return carry

                lax.fori_loop(0, _NB // _L, zero_hist, 0, unroll=True)

                def sweep(c, carry):
                    base = c * _L
                    cx = px_v[pl.ds(base, _L)]
                    cy = py_v[pl.ds(base, _L)]
                    cz = pz_v[pl.ds(base, _L)]
                    dx = qx - cx
                    dy = qy - cy
                    dz = qz - cz
                    d = dx * dx + dy * dy + dz * dz

                    fx = fx_v[pl.ds(base, _L)]
                    fy = fy_v[pl.ds(base, _L)]
                    fz = fz_v[pl.ds(base, _L)]
                    gx = qfx - fx
                    gy = qfy - fy
                    gz = qfz - fz
                    sqf = gx * gx + gy * gy + gz * gz

                    sqfs = jnp.maximum(sqf, jnp.float32(1e-12))
                    i0 = plsc.bitcast(sqfs, jnp.int32)
                    i1 = jnp.int32(0x5F3759DF) - lax.shift_right_logical(i0, 1)
                    y = plsc.bitcast(i1, jnp.float32)
                    half = jnp.float32(0.5) * sqfs
                    for _ in range(3):
                        y = y * (jnp.float32(1.5) - half * y * y)
                    F = jnp.where(sqf > 0.0, sqfs * y, jnp.float32(0.0))

                    mask = d <= 1.0
                    bucket = jnp.minimum(
                        (d * jnp.float32(_NB)).astype(jnp.int32), _NB - 1)
                    plsc.addupdate_scatter(hist, [bucket], ones_v, mask=mask)
                    plsc.addupdate_scatter(fhist, [bucket], F, mask=mask)
                    return carry

                lax.fori_loop(0, _N // _L, sweep, 0, unroll=8)

                def select(i, carry):
                    run, acc = carry
                    h = hist[pl.ds(i * _L, _L)]
                    fh = fhist[pl.ds(i * _L, _L)]
                    incl = plsc.cumsum(h)
                    excl = incl - h + run
                    w = jnp.clip((kf - excl) / jnp.maximum(h, 1.0), 0.0, 1.0)
                    acc = acc + w * fh
                    run = run + jnp.sum(h)
                    return run, acc

                _, acc = lax.fori_loop(0, _NB // _L, select,
                                       (jnp.float32(0.0), acc), unroll=4)
                return acc

            acc = lax.fori_loop(0, _QPW, per_query, acc)

        out_v[...] = acc
        pltpu.sync_copy(out_v, out_hbm.at[wid])

    return sc_knn


def kernel(pc_source, pred_flow):
    soa = jnp.concatenate(
        [jnp.transpose(pc_source, (0, 2, 1)),
         jnp.transpose(pred_flow, (0, 2, 1))], axis=1)
    soa = soa.reshape(_B * 6, _N)
    out = _make()(soa)
    return jnp.sum(out) / jnp.float32(_B * _N * _K)
